# baseline (speedup 1.0000x reference)
import functools

import jax
import jax.numpy as jnp
from jax.experimental import pallas as pl
from jax.experimental.pallas import tpu as pltpu

_LOG2E = 1.4426950408889634


def _proj_body(h_ref, w_ref, asrc_ref, adst_ref,
               whp_ref, fsrc_ref, d1_ref, d2_ref, mmax_ref,
               *, apply_elu, block):
    a = pl.program_id(0)
    h = h_ref[...]
    if apply_elu:
        h = jnp.where(h > 0, h, jnp.exp(h) - jnp.float32(1.0))
    wh = jnp.dot(h, w_ref[...], preferred_element_type=jnp.float32)
    f = wh.shape[1]
    ones_col = (jax.lax.broadcasted_iota(jnp.int32, (1, f), 1) == 0)
    whp_ref[...] = jnp.concatenate(
        [wh.astype(jnp.bfloat16),
         jnp.broadcast_to(ones_col.astype(jnp.bfloat16), wh.shape)],
        axis=1)
    fsrc_ref[...] = jnp.dot(wh, asrc_ref[...],
                            preferred_element_type=jnp.float32)
    fd = jax.lax.dot_general(adst_ref[...], wh, (((0,), (1,)), ((), ())),
                             preferred_element_type=jnp.float32)
    d1_ref[...] = (jnp.float32(_LOG2E) * fd)[None]
    d2_ref[...] = (jnp.float32(0.2 * _LOG2E) * fd)[None]
    bmax = jnp.max(fd)

    @pl.when(a == 0)
    def _first():
        mmax_ref[0, 0] = bmax

    @pl.when(a > 0)
    def _rest():
        mmax_ref[0, 0] = jnp.maximum(mmax_ref[0, 0], bmax)


def _project(h, w, a_src, a_dst, apply_elu, block):
    n, f = h.shape
    return pl.pallas_call(
        functools.partial(_proj_body, apply_elu=apply_elu, block=block),
        grid=(n // block,),
        in_specs=[
            pl.BlockSpec((block, f), lambda a: (a, 0)),
            pl.BlockSpec((f, f), lambda a: (0, 0)),
            pl.BlockSpec((f, 1), lambda a: (0, 0)),
            pl.BlockSpec((f, 1), lambda a: (0, 0)),
        ],
        out_specs=[
            pl.BlockSpec((block, 2 * f), lambda a: (a, 0)),
            pl.BlockSpec((block, 1), lambda a: (a, 0)),
            pl.BlockSpec((1, 1, block), lambda a: (a, 0, 0)),
            pl.BlockSpec((1, 1, block), lambda a: (a, 0, 0)),
            pl.BlockSpec(memory_space=pltpu.SMEM),
        ],
        out_shape=[
            jax.ShapeDtypeStruct((n, 2 * f), jnp.bfloat16),
            jax.ShapeDtypeStruct((n, 1), jnp.float32),
            jax.ShapeDtypeStruct((n // block, 1, block), jnp.float32),
            jax.ShapeDtypeStruct((n // block, 1, block), jnp.float32),
            jax.ShapeDtypeStruct((1, 1), jnp.float32),
        ],
        compiler_params=pltpu.CompilerParams(
            dimension_semantics=("arbitrary",)),
    )(h, w, a_src, a_dst)


def _attn_body(mmax_ref, adj_ref, whp_ref, fsrc_ref, d1_ref, d2_ref,
               out_ref):
    f = out_ref.shape[1]
    c = jnp.float32(_LOG2E)
    fsrc = fsrc_ref[...]
    v = fsrc + mmax_ref[0, 0]
    cm = c * jnp.maximum(v, jnp.float32(0.2) * v)
    row_a = c * fsrc - cm
    row_c = jnp.float32(0.2) * c * fsrc - cm
    t1 = row_a + d1_ref[...]
    t2 = row_c + d2_ref[...]
    t3 = jnp.maximum(t1, t2)
    t3 = jnp.where(adj_ref[...] > 0, t3, jnp.float32(-1e9))
    p = jnp.exp2(t3).astype(jnp.bfloat16)
    acc = jnp.dot(p, whp_ref[...], preferred_element_type=jnp.float32)
    out_ref[...] = acc[:, :f] / acc[:, f:f + 1]


def _attention(adj, whp, fsrc, d1, d2, mmax, bi):
    n = adj.shape[0]
    f = whp.shape[1] // 2
    d1 = d1.reshape(1, n)
    d2 = d2.reshape(1, n)
    return pl.pallas_call(
        _attn_body,
        grid=(n // bi,),
        in_specs=[
            pl.BlockSpec(memory_space=pltpu.SMEM),
            pl.BlockSpec((bi, n), lambda i: (i, 0)),
            pl.BlockSpec((n, 2 * f), lambda i: (0, 0)),
            pl.BlockSpec((bi, 1), lambda i: (i, 0)),
            pl.BlockSpec((1, n), lambda i: (0, 0)),
            pl.BlockSpec((1, n), lambda i: (0, 0)),
        ],
        out_specs=pl.BlockSpec((bi, f), lambda i: (i, 0)),
        out_shape=jax.ShapeDtypeStruct((n, f), jnp.float32),
        compiler_params=pltpu.CompilerParams(
            dimension_semantics=("arbitrary",)),
    )(mmax, adj, whp, fsrc, d1, d2)


def _pick(n, prefs):
    for p in prefs:
        if n % p == 0:
            return p
    return n


def kernel(x, adj, W1, a1_src, a1_dst, W2, a2_src, a2_dst):
    n, _ = x.shape
    ba = _pick(n, (2000, 1000, 500))
    bi = _pick(n, (400, 200, 100))

    a1s = a1_src.reshape(-1, 1).astype(jnp.float32)
    a1d = a1_dst.reshape(-1, 1).astype(jnp.float32)
    a2s = a2_src.reshape(-1, 1).astype(jnp.float32)
    a2d = a2_dst.reshape(-1, 1).astype(jnp.float32)

    whp, fs, d1, d2, mm = _project(x, W1, a1s, a1d, False, ba)
    h1 = _attention(adj, whp, fs, d1, d2, mm, bi)
    whp, fs, d1, d2, mm = _project(h1, W2, a2s, a2d, True, ba)
    return _attention(adj, whp, fs, d1, d2, mm, bi)


# speedup vs baseline: 1.0220x; 1.0220x over previous
import functools

import jax
import jax.numpy as jnp
from jax.experimental import pallas as pl
from jax.experimental.pallas import tpu as pltpu

_LOG2E = 1.4426950408889634


def _proj_body(h_ref, w_ref, asrc_ref, adst_ref,
               whp_ref, fsrc_ref, d1_ref, d2_ref, mmax_ref,
               *, apply_elu, block):
    a = pl.program_id(0)
    h = h_ref[...]
    if apply_elu:
        h = jnp.where(h > 0, h, jnp.exp(h) - jnp.float32(1.0))
    wh = jnp.dot(h, w_ref[...], preferred_element_type=jnp.float32)
    f = wh.shape[1]
    ones_col = (jax.lax.broadcasted_iota(jnp.int32, (1, f), 1) == 0)
    whp_ref[...] = jnp.concatenate(
        [wh.astype(jnp.bfloat16),
         jnp.broadcast_to(ones_col.astype(jnp.bfloat16), wh.shape)],
        axis=1)
    fsrc_ref[...] = jnp.dot(wh, asrc_ref[...],
                            preferred_element_type=jnp.float32)
    fd = jax.lax.dot_general(adst_ref[...], wh, (((0,), (1,)), ((), ())),
                             preferred_element_type=jnp.float32)
    d1_ref[...] = (jnp.float32(_LOG2E) * fd)[None]
    d2_ref[...] = (jnp.float32(0.2 * _LOG2E) * fd)[None]
    bmax = jnp.max(fd)

    @pl.when(a == 0)
    def _first():
        mmax_ref[0, 0] = bmax

    @pl.when(a > 0)
    def _rest():
        mmax_ref[0, 0] = jnp.maximum(mmax_ref[0, 0], bmax)


def _project(h, w, a_src, a_dst, apply_elu, block):
    n, f = h.shape
    return pl.pallas_call(
        functools.partial(_proj_body, apply_elu=apply_elu, block=block),
        grid=(n // block,),
        in_specs=[
            pl.BlockSpec((block, f), lambda a: (a, 0)),
            pl.BlockSpec((f, f), lambda a: (0, 0)),
            pl.BlockSpec((f, 1), lambda a: (0, 0)),
            pl.BlockSpec((f, 1), lambda a: (0, 0)),
        ],
        out_specs=[
            pl.BlockSpec((block, 2 * f), lambda a: (a, 0)),
            pl.BlockSpec((block, 1), lambda a: (a, 0)),
            pl.BlockSpec((1, 1, block), lambda a: (a, 0, 0)),
            pl.BlockSpec((1, 1, block), lambda a: (a, 0, 0)),
            pl.BlockSpec(memory_space=pltpu.SMEM),
        ],
        out_shape=[
            jax.ShapeDtypeStruct((n, 2 * f), jnp.bfloat16),
            jax.ShapeDtypeStruct((n, 1), jnp.float32),
            jax.ShapeDtypeStruct((n // block, 1, block), jnp.float32),
            jax.ShapeDtypeStruct((n // block, 1, block), jnp.float32),
            jax.ShapeDtypeStruct((1, 1), jnp.float32),
        ],
        compiler_params=pltpu.CompilerParams(
            dimension_semantics=("arbitrary",)),
    )(h, w, a_src, a_dst)


def _rows(mmax_ref, fsrc_ref):
    c = jnp.float32(_LOG2E)
    fsrc = fsrc_ref[...]
    v = fsrc + mmax_ref[0, 0]
    cm = c * jnp.maximum(v, jnp.float32(0.2) * v)
    row_a = c * fsrc - cm
    row_c = jnp.float32(0.2) * c * fsrc - cm
    return row_a, row_c


def _attn_pack_body(mmax_ref, adj_ref, whp_ref, fsrc_ref, d1_ref, d2_ref,
                    out_ref, pk_ref, *, nbits, cw):
    f = out_ref.shape[1]
    row_a, row_c = _rows(mmax_ref, fsrc_ref)
    adj = adj_ref[...]
    t1 = row_a + d1_ref[...]
    t2 = row_c + d2_ref[...]
    t3 = jnp.maximum(t1, t2)
    t3 = jnp.where(adj > 0, t3, jnp.float32(-1e9))
    p = jnp.exp2(t3).astype(jnp.bfloat16)
    acc = jnp.dot(p, whp_ref[...], preferred_element_type=jnp.float32)
    out_ref[...] = acc[:, :f] / acc[:, f:f + 1]
    w = adj[:, 0:cw]
    for s in range(1, nbits):
        w = w | jnp.left_shift(adj[:, s * cw:(s + 1) * cw], s)
    pk_ref[...] = w


def _attn_packed_body(mmax_ref, pk_ref, whp_ref, fsrc_ref, d1_ref, d2_ref,
                      out_ref, *, nbits, cw):
    f = out_ref.shape[1]
    row_a, row_c = _rows(mmax_ref, fsrc_ref)
    w = pk_ref[...]
    parts = []
    for s in range(nbits):
        t1 = row_a + d1_ref[:, s * cw:(s + 1) * cw]
        t2 = row_c + d2_ref[:, s * cw:(s + 1) * cw]
        t3 = jnp.maximum(t1, t2)
        t3 = jnp.where((w & (1 << s)) > 0, t3, jnp.float32(-1e9))
        parts.append(jnp.exp2(t3).astype(jnp.bfloat16))
    p = jnp.concatenate(parts, axis=1)
    acc = jnp.dot(p, whp_ref[...], preferred_element_type=jnp.float32)
    out_ref[...] = acc[:, :f] / acc[:, f:f + 1]


_NBITS = 16


def _attention(adj_or_pk, whp, fsrc, d1, d2, mmax, bi, packed):
    n = fsrc.shape[0]
    f = whp.shape[1] // 2
    npad = whp.shape[0]
    cw = npad // _NBITS
    pad = ((0, 0), (0, npad - n))
    d1 = jnp.pad(d1.reshape(1, n), pad, constant_values=-1e9)
    d2 = jnp.pad(d2.reshape(1, n), pad, constant_values=-1e9)
    if packed:
        body = functools.partial(_attn_packed_body, nbits=_NBITS, cw=cw)
        adj_spec = pl.BlockSpec((bi, cw), lambda i: (i, 0))
        out_shape = jax.ShapeDtypeStruct((n, f), jnp.float32)
        out_specs = pl.BlockSpec((bi, f), lambda i: (i, 0))
    else:
        body = functools.partial(_attn_pack_body, nbits=_NBITS, cw=cw)
        adj_spec = pl.BlockSpec((bi, npad), lambda i: (i, 0))
        out_shape = [jax.ShapeDtypeStruct((n, f), jnp.float32),
                     jax.ShapeDtypeStruct((n, cw), jnp.int32)]
        out_specs = [pl.BlockSpec((bi, f), lambda i: (i, 0)),
                     pl.BlockSpec((bi, cw), lambda i: (i, 0))]
    return pl.pallas_call(
        body,
        grid=(n // bi,),
        in_specs=[
            pl.BlockSpec(memory_space=pltpu.SMEM),
            adj_spec,
            pl.BlockSpec((npad, 2 * f), lambda i: (0, 0)),
            pl.BlockSpec((bi, 1), lambda i: (i, 0)),
            pl.BlockSpec((1, npad), lambda i: (0, 0)),
            pl.BlockSpec((1, npad), lambda i: (0, 0)),
        ],
        out_specs=out_specs,
        out_shape=out_shape,
        compiler_params=pltpu.CompilerParams(
            dimension_semantics=("arbitrary",)),
    )(mmax, adj_or_pk, whp, fsrc, d1, d2)


def _pick(n, prefs):
    for p in prefs:
        if n % p == 0:
            return p
    return n


def kernel(x, adj, W1, a1_src, a1_dst, W2, a2_src, a2_dst):
    n, _ = x.shape
    ba = _pick(n, (2000, 1000, 500))
    bi = _pick(n, (400, 200, 100))

    a1s = a1_src.reshape(-1, 1).astype(jnp.float32)
    a1d = a1_dst.reshape(-1, 1).astype(jnp.float32)
    a2s = a2_src.reshape(-1, 1).astype(jnp.float32)
    a2d = a2_dst.reshape(-1, 1).astype(jnp.float32)

    npad = -(-n // (128 * _NBITS)) * (128 * _NBITS)

    whp, fs, d1, d2, mm = _project(x, W1, a1s, a1d, False, ba)
    whp = jnp.pad(whp, ((0, npad - n), (0, 0)))
    h1, pk = _attention(adj, whp, fs, d1, d2, mm, bi, packed=False)
    whp, fs, d1, d2, mm = _project(h1, W2, a2s, a2d, True, ba)
    whp = jnp.pad(whp, ((0, npad - n), (0, 0)))
    return _attention(pk, whp, fs, d1, d2, mm, bi, packed=True)


# speedup vs baseline: 1.0586x; 1.0358x over previous
import functools

import jax
import jax.numpy as jnp
from jax.experimental import pallas as pl
from jax.experimental.pallas import tpu as pltpu

_LOG2E = 1.4426950408889634


def _proj_body(h_ref, w_ref, asrc_ref, adst_ref,
               whp_ref, fsrc_ref, d1_ref, d2_ref, mmax_ref,
               *, apply_elu, block, n):
    a = pl.program_id(0)
    h = h_ref[...]
    if apply_elu:
        h = jnp.where(h > 0, h, jnp.exp(h) - jnp.float32(1.0))
    wh = jnp.dot(h, w_ref[...], preferred_element_type=jnp.float32)
    f = wh.shape[1]
    rid = a * block + jax.lax.broadcasted_iota(jnp.int32, (block, 1), 0)
    rmask = rid < n
    wh = jnp.where(rmask, wh, jnp.float32(0.0))
    ones_col = (jax.lax.broadcasted_iota(jnp.int32, (1, f), 1) == 0)
    whp_ref[...] = jnp.concatenate(
        [wh.astype(jnp.bfloat16),
         jnp.where(jnp.logical_and(rmask, ones_col), jnp.float32(1.0),
                   jnp.float32(0.0)).astype(jnp.bfloat16)],
        axis=1)
    fsrc_ref[...] = jnp.dot(wh, asrc_ref[...],
                            preferred_element_type=jnp.float32)
    fd = jax.lax.dot_general(adst_ref[...], wh, (((0,), (1,)), ((), ())),
                             preferred_element_type=jnp.float32)
    cid = a * block + jax.lax.broadcasted_iota(jnp.int32, (1, block), 1)
    cmask = cid < n
    fd = jnp.where(cmask, fd, jnp.float32(-1e9))
    d1_ref[:, pl.ds(a * block, block)] = jnp.float32(_LOG2E) * fd
    d2_ref[:, pl.ds(a * block, block)] = jnp.float32(0.2 * _LOG2E) * fd
    bmax = jnp.max(fd)

    @pl.when(a == 0)
    def _first():
        mmax_ref[0, 0] = bmax

    @pl.when(a > 0)
    def _rest():
        mmax_ref[0, 0] = jnp.maximum(mmax_ref[0, 0], bmax)


def _project(h, w, a_src, a_dst, apply_elu, block, npad):
    n, f = h.shape
    return pl.pallas_call(
        functools.partial(_proj_body, apply_elu=apply_elu, block=block, n=n),
        grid=(npad // block,),
        in_specs=[
            pl.BlockSpec((block, f), lambda a: (a, 0)),
            pl.BlockSpec((f, f), lambda a: (0, 0)),
            pl.BlockSpec((f, 1), lambda a: (0, 0)),
            pl.BlockSpec((f, 1), lambda a: (0, 0)),
        ],
        out_specs=[
            pl.BlockSpec((block, 2 * f), lambda a: (a, 0)),
            pl.BlockSpec((block, 1), lambda a: (a, 0)),
            pl.BlockSpec((1, npad), lambda a: (0, 0)),
            pl.BlockSpec((1, npad), lambda a: (0, 0)),
            pl.BlockSpec(memory_space=pltpu.SMEM),
        ],
        out_shape=[
            jax.ShapeDtypeStruct((npad, 2 * f), jnp.bfloat16),
            jax.ShapeDtypeStruct((npad, 1), jnp.float32),
            jax.ShapeDtypeStruct((1, npad), jnp.float32),
            jax.ShapeDtypeStruct((1, npad), jnp.float32),
            jax.ShapeDtypeStruct((1, 1), jnp.float32),
        ],
        compiler_params=pltpu.CompilerParams(
            dimension_semantics=("arbitrary",)),
    )(h, w, a_src, a_dst)


def _rows(mmax_ref, fsrc_ref):
    c = jnp.float32(_LOG2E)
    fsrc = fsrc_ref[...]
    v = fsrc + mmax_ref[0, 0]
    cm = c * jnp.maximum(v, jnp.float32(0.2) * v)
    row_a = c * fsrc - cm
    row_c = jnp.float32(0.2) * c * fsrc - cm
    return row_a, row_c


def _attn_pack_body(mmax_ref, adj_ref, whp_ref, fsrc_ref, d1_ref, d2_ref,
                    out_ref, pk_ref, *, nbits, cw):
    f = out_ref.shape[1]
    row_a, row_c = _rows(mmax_ref, fsrc_ref)
    adj = adj_ref[...]
    t1 = row_a + d1_ref[...]
    t2 = row_c + d2_ref[...]
    t3 = jnp.maximum(t1, t2)
    t3 = jnp.where(adj > 0, t3, jnp.float32(-1e9))
    p = jnp.exp2(t3).astype(jnp.bfloat16)
    acc = jnp.dot(p, whp_ref[...], preferred_element_type=jnp.float32)
    out_ref[...] = acc[:, :f] / acc[:, f:f + 1]
    w = adj[:, 0:cw]
    for s in range(1, nbits):
        w = w | jnp.left_shift(adj[:, s * cw:(s + 1) * cw], s)
    pk_ref[...] = w


def _attn_packed_body(mmax_ref, pk_ref, whp_ref, fsrc_ref, d1_ref, d2_ref,
                      out_ref, *, nbits, cw):
    f = out_ref.shape[1]
    row_a, row_c = _rows(mmax_ref, fsrc_ref)
    w = pk_ref[...]
    parts = []
    for s in range(nbits):
        t1 = row_a + d1_ref[:, s * cw:(s + 1) * cw]
        t2 = row_c + d2_ref[:, s * cw:(s + 1) * cw]
        t3 = jnp.maximum(t1, t2)
        t3 = jnp.where((w & (1 << s)) > 0, t3, jnp.float32(-1e9))
        parts.append(jnp.exp2(t3).astype(jnp.bfloat16))
    p = jnp.concatenate(parts, axis=1)
    acc = jnp.dot(p, whp_ref[...], preferred_element_type=jnp.float32)
    out_ref[...] = acc[:, :f] / acc[:, f:f + 1]


_NBITS = 16


def _attention(adj_or_pk, whp, fsrc, d1, d2, mmax, n, bi, packed):
    f = whp.shape[1] // 2
    npad = whp.shape[0]
    cw = npad // _NBITS
    if packed:
        body = functools.partial(_attn_packed_body, nbits=_NBITS, cw=cw)
        adj_spec = pl.BlockSpec((bi, cw), lambda i: (i, 0))
        out_shape = jax.ShapeDtypeStruct((n, f), jnp.float32)
        out_specs = pl.BlockSpec((bi, f), lambda i: (i, 0))
    else:
        body = functools.partial(_attn_pack_body, nbits=_NBITS, cw=cw)
        adj_spec = pl.BlockSpec((bi, npad), lambda i: (i, 0))
        out_shape = [jax.ShapeDtypeStruct((n, f), jnp.float32),
                     jax.ShapeDtypeStruct((n, cw), jnp.int32)]
        out_specs = [pl.BlockSpec((bi, f), lambda i: (i, 0)),
                     pl.BlockSpec((bi, cw), lambda i: (i, 0))]
    return pl.pallas_call(
        body,
        grid=(n // bi,),
        in_specs=[
            pl.BlockSpec(memory_space=pltpu.SMEM),
            adj_spec,
            pl.BlockSpec((npad, 2 * f), lambda i: (0, 0)),
            pl.BlockSpec((bi, 1), lambda i: (i, 0)),
            pl.BlockSpec((1, npad), lambda i: (0, 0)),
            pl.BlockSpec((1, npad), lambda i: (0, 0)),
        ],
        out_specs=out_specs,
        out_shape=out_shape,
        compiler_params=pltpu.CompilerParams(
            dimension_semantics=("arbitrary",)),
    )(mmax, adj_or_pk, whp, fsrc, d1, d2)


def _pick(n, prefs):
    for p in prefs:
        if n % p == 0:
            return p
    return n


def kernel(x, adj, W1, a1_src, a1_dst, W2, a2_src, a2_dst):
    n, _ = x.shape
    bi = _pick(n, (400, 200, 100))
    npad = -(-n // (128 * _NBITS)) * (128 * _NBITS)
    ba = 2048 if npad % 2048 == 0 else npad

    a1s = a1_src.reshape(-1, 1).astype(jnp.float32)
    a1d = a1_dst.reshape(-1, 1).astype(jnp.float32)
    a2s = a2_src.reshape(-1, 1).astype(jnp.float32)
    a2d = a2_dst.reshape(-1, 1).astype(jnp.float32)

    whp, fs, d1, d2, mm = _project(x, W1, a1s, a1d, False, ba, npad)
    h1, pk = _attention(adj, whp, fs, d1, d2, mm, n, bi, packed=False)
    whp, fs, d1, d2, mm = _project(h1, W2, a2s, a2d, True, ba, npad)
    return _attention(pk, whp, fs, d1, d2, mm, n, bi, packed=True)


# speedup vs baseline: 1.2131x; 1.1460x over previous
import functools

import jax
import jax.numpy as jnp
from jax.experimental import pallas as pl
from jax.experimental.pallas import tpu as pltpu

_LOG2E = 1.4426950408889634


def _proj_body(h_ref, w_ref, asrc_ref, adst_ref,
               whp_ref, fsrc_ref, d1_ref, d2_ref, mmax_ref,
               *, apply_elu, block, n):
    a = pl.program_id(0)
    h = h_ref[...]
    if apply_elu:
        h = jnp.where(h > 0, h, jnp.exp(h) - jnp.float32(1.0))
    wh = jnp.dot(h, w_ref[...], preferred_element_type=jnp.float32)
    f = wh.shape[1]
    rid = a * block + jax.lax.broadcasted_iota(jnp.int32, (block, 1), 0)
    rmask = rid < n
    wh = jnp.where(rmask, wh, jnp.float32(0.0))
    ones_col = (jax.lax.broadcasted_iota(jnp.int32, (1, f), 1) == 0)
    whp_ref[...] = jnp.concatenate(
        [wh.astype(jnp.bfloat16),
         jnp.where(jnp.logical_and(rmask, ones_col), jnp.float32(1.0),
                   jnp.float32(0.0)).astype(jnp.bfloat16)],
        axis=1)
    fsrc_ref[...] = jnp.dot(wh, asrc_ref[...],
                            preferred_element_type=jnp.float32)
    fd = jax.lax.dot_general(adst_ref[...], wh, (((0,), (1,)), ((), ())),
                             preferred_element_type=jnp.float32)
    cid = a * block + jax.lax.broadcasted_iota(jnp.int32, (1, block), 1)
    cmask = cid < n
    fd = jnp.where(cmask, fd, jnp.float32(-1e9))
    d1_ref[:, pl.ds(a * block, block)] = jnp.float32(_LOG2E) * fd
    d2_ref[:, pl.ds(a * block, block)] = jnp.float32(0.2 * _LOG2E) * fd
    bmax = jnp.max(fd)

    @pl.when(a == 0)
    def _first():
        mmax_ref[0, 0] = bmax

    @pl.when(a > 0)
    def _rest():
        mmax_ref[0, 0] = jnp.maximum(mmax_ref[0, 0], bmax)


def _project(h, w, a_src, a_dst, apply_elu, block, npad):
    n, f = h.shape
    return pl.pallas_call(
        functools.partial(_proj_body, apply_elu=apply_elu, block=block, n=n),
        grid=(npad // block,),
        in_specs=[
            pl.BlockSpec((block, f), lambda a: (a, 0)),
            pl.BlockSpec((f, f), lambda a: (0, 0)),
            pl.BlockSpec((f, 1), lambda a: (0, 0)),
            pl.BlockSpec((f, 1), lambda a: (0, 0)),
        ],
        out_specs=[
            pl.BlockSpec((block, 2 * f), lambda a: (a, 0)),
            pl.BlockSpec((block, 1), lambda a: (a, 0)),
            pl.BlockSpec((1, npad), lambda a: (0, 0)),
            pl.BlockSpec((1, npad), lambda a: (0, 0)),
            pl.BlockSpec(memory_space=pltpu.SMEM),
        ],
        out_shape=[
            jax.ShapeDtypeStruct((npad, 2 * f), jnp.bfloat16),
            jax.ShapeDtypeStruct((npad, 1), jnp.float32),
            jax.ShapeDtypeStruct((1, npad), jnp.float32),
            jax.ShapeDtypeStruct((1, npad), jnp.float32),
            jax.ShapeDtypeStruct((1, 1), jnp.float32),
        ],
        compiler_params=pltpu.CompilerParams(
            dimension_semantics=("arbitrary",)),
    )(h, w, a_src, a_dst)


def _rows(mmax_ref, fsrc_ref):
    c = jnp.float32(_LOG2E)
    fsrc = fsrc_ref[...]
    v = fsrc + mmax_ref[0, 0]
    cm = c * jnp.maximum(v, jnp.float32(0.2) * v)
    row_a = c * fsrc - cm
    row_c = jnp.float32(0.2) * c * fsrc - cm
    return row_a, row_c


def _attn_pack_body(mmax_ref, adj_ref, whp_ref, fsrc_ref, d1_ref, d2_ref,
                    out_ref, pk_ref, *, nbits, cw):
    f = out_ref.shape[1]
    row_a, row_c = _rows(mmax_ref, fsrc_ref)
    adj = adj_ref[...]
    t1 = row_a + d1_ref[...]
    t2 = row_c + d2_ref[...]
    t3 = jnp.maximum(t1, t2)
    t3 = jnp.where(adj > 0, t3, jnp.float32(-1e9))
    p = jnp.exp2(t3).astype(jnp.bfloat16)
    acc = jnp.dot(p, whp_ref[...], preferred_element_type=jnp.float32)
    out_ref[...] = acc[:, :f] / acc[:, f:f + 1]
    w = adj[:, 0:cw]
    for s in range(1, nbits):
        w = w | jnp.left_shift(adj[:, s * cw:(s + 1) * cw], s)
    pk_ref[...] = w


def _attn_packed_body(mmax_ref, pk_ref, whp_ref, fsrc_ref, d1_ref, d2_ref,
                      out_ref, *, nbits, cw):
    f = out_ref.shape[1]
    row_a, row_c = _rows(mmax_ref, fsrc_ref)
    row_a = row_a.astype(jnp.bfloat16)
    row_c = row_c.astype(jnp.bfloat16)
    w = pk_ref[...]
    parts = []
    for s in range(nbits):
        t1 = row_a + d1_ref[:, s * cw:(s + 1) * cw].astype(jnp.bfloat16)
        t2 = row_c + d2_ref[:, s * cw:(s + 1) * cw].astype(jnp.bfloat16)
        t3 = jnp.maximum(t1, t2)
        t3 = jnp.where((w & (1 << s)) > 0, t3, jnp.bfloat16(-1e9))
        parts.append(jnp.exp2(t3))
    p = jnp.concatenate(parts, axis=1)
    acc = jnp.dot(p, whp_ref[...], preferred_element_type=jnp.float32)
    out_ref[...] = acc[:, :f] / acc[:, f:f + 1]


_NBITS = 16


def _attention(adj_or_pk, whp, fsrc, d1, d2, mmax, n, bi, packed):
    f = whp.shape[1] // 2
    npad = whp.shape[0]
    cw = npad // _NBITS
    if packed:
        body = functools.partial(_attn_packed_body, nbits=_NBITS, cw=cw)
        adj_spec = pl.BlockSpec((bi, cw), lambda i: (i, 0))
        out_shape = jax.ShapeDtypeStruct((n, f), jnp.float32)
        out_specs = pl.BlockSpec((bi, f), lambda i: (i, 0))
    else:
        body = functools.partial(_attn_pack_body, nbits=_NBITS, cw=cw)
        adj_spec = pl.BlockSpec((bi, npad), lambda i: (i, 0))
        out_shape = [jax.ShapeDtypeStruct((n, f), jnp.float32),
                     jax.ShapeDtypeStruct((n, cw), jnp.int32)]
        out_specs = [pl.BlockSpec((bi, f), lambda i: (i, 0)),
                     pl.BlockSpec((bi, cw), lambda i: (i, 0))]
    return pl.pallas_call(
        body,
        grid=(n // bi,),
        in_specs=[
            pl.BlockSpec(memory_space=pltpu.SMEM),
            adj_spec,
            pl.BlockSpec((npad, 2 * f), lambda i: (0, 0)),
            pl.BlockSpec((bi, 1), lambda i: (i, 0)),
            pl.BlockSpec((1, npad), lambda i: (0, 0)),
            pl.BlockSpec((1, npad), lambda i: (0, 0)),
        ],
        out_specs=out_specs,
        out_shape=out_shape,
        compiler_params=pltpu.CompilerParams(
            dimension_semantics=("arbitrary",)),
    )(mmax, adj_or_pk, whp, fsrc, d1, d2)


def _pick(n, prefs):
    for p in prefs:
        if n % p == 0:
            return p
    return n


def kernel(x, adj, W1, a1_src, a1_dst, W2, a2_src, a2_dst):
    n, _ = x.shape
    bi = _pick(n, (400, 200, 100))
    npad = -(-n // (128 * _NBITS)) * (128 * _NBITS)
    ba = 2048 if npad % 2048 == 0 else npad

    a1s = a1_src.reshape(-1, 1).astype(jnp.float32)
    a1d = a1_dst.reshape(-1, 1).astype(jnp.float32)
    a2s = a2_src.reshape(-1, 1).astype(jnp.float32)
    a2d = a2_dst.reshape(-1, 1).astype(jnp.float32)

    whp, fs, d1, d2, mm = _project(x, W1, a1s, a1d, False, ba, npad)
    h1, pk = _attention(adj, whp, fs, d1, d2, mm, n, bi, packed=False)
    whp, fs, d1, d2, mm = _project(h1, W2, a2s, a2d, True, ba, npad)
    return _attention(pk, whp, fs, d1, d2, mm, n, bi, packed=True)


# speedup vs baseline: 1.2624x; 1.0406x over previous
import functools

import jax
import jax.numpy as jnp
from jax.experimental import pallas as pl
from jax.experimental.pallas import tpu as pltpu

_LOG2E = 1.4426950408889634


def _proj_body(h_ref, w_ref, asrc_ref, adst_ref,
               whp_ref, fsrc_ref, d1_ref, d2_ref, mmax_ref,
               *, apply_elu, block, n):
    a = pl.program_id(0)
    h = h_ref[...]
    if apply_elu:
        h = jnp.where(h > 0, h, jnp.exp(h) - jnp.float32(1.0))
    wh = jnp.dot(h, w_ref[...], preferred_element_type=jnp.float32)
    f = wh.shape[1]
    rid = a * block + jax.lax.broadcasted_iota(jnp.int32, (block, 1), 0)
    rmask = rid < n
    wh = jnp.where(rmask, wh, jnp.float32(0.0))
    ones_col = (jax.lax.broadcasted_iota(jnp.int32, (1, f), 1) == 0)
    whp_ref[...] = jnp.concatenate(
        [wh.astype(jnp.bfloat16),
         jnp.where(jnp.logical_and(rmask, ones_col), jnp.float32(1.0),
                   jnp.float32(0.0)).astype(jnp.bfloat16)],
        axis=1)
    fsrc_ref[...] = jnp.dot(wh, asrc_ref[...],
                            preferred_element_type=jnp.float32)
    fd = jax.lax.dot_general(adst_ref[...], wh, (((0,), (1,)), ((), ())),
                             preferred_element_type=jnp.float32)
    cid = a * block + jax.lax.broadcasted_iota(jnp.int32, (1, block), 1)
    cmask = cid < n
    fd = jnp.where(cmask, fd, jnp.float32(-1e9))
    d1_ref[:, pl.ds(a * block, block)] = jnp.float32(_LOG2E) * fd
    d2_ref[:, pl.ds(a * block, block)] = jnp.float32(0.2 * _LOG2E) * fd
    bmax = jnp.max(fd)

    @pl.when(a == 0)
    def _first():
        mmax_ref[0, 0] = bmax

    @pl.when(a > 0)
    def _rest():
        mmax_ref[0, 0] = jnp.maximum(mmax_ref[0, 0], bmax)


def _project(h, w, a_src, a_dst, apply_elu, block, npad):
    n, f = h.shape
    return pl.pallas_call(
        functools.partial(_proj_body, apply_elu=apply_elu, block=block, n=n),
        grid=(npad // block,),
        in_specs=[
            pl.BlockSpec((block, f), lambda a: (a, 0)),
            pl.BlockSpec((f, f), lambda a: (0, 0)),
            pl.BlockSpec((f, 1), lambda a: (0, 0)),
            pl.BlockSpec((f, 1), lambda a: (0, 0)),
        ],
        out_specs=[
            pl.BlockSpec((block, 2 * f), lambda a: (a, 0)),
            pl.BlockSpec((block, 1), lambda a: (a, 0)),
            pl.BlockSpec((1, npad), lambda a: (0, 0)),
            pl.BlockSpec((1, npad), lambda a: (0, 0)),
            pl.BlockSpec(memory_space=pltpu.SMEM),
        ],
        out_shape=[
            jax.ShapeDtypeStruct((npad, 2 * f), jnp.bfloat16),
            jax.ShapeDtypeStruct((npad, 1), jnp.float32),
            jax.ShapeDtypeStruct((1, npad), jnp.float32),
            jax.ShapeDtypeStruct((1, npad), jnp.float32),
            jax.ShapeDtypeStruct((1, 1), jnp.float32),
        ],
        compiler_params=pltpu.CompilerParams(
            dimension_semantics=("arbitrary",)),
    )(h, w, a_src, a_dst)


def _rows(mmax_ref, fsrc_ref):
    c = jnp.float32(_LOG2E)
    fsrc = fsrc_ref[...]
    v = fsrc + mmax_ref[0, 0]
    cm = c * jnp.maximum(v, jnp.float32(0.2) * v)
    row_a = c * fsrc - cm
    row_c = jnp.float32(0.2) * c * fsrc - cm
    return row_a, row_c


def _attn_pack_body(mmax_ref, adjl_ref, adjr_ref, whp_ref, fsrc_ref,
                    d1_ref, d2_ref, out_ref, pk_ref, *, nbits, cw):
    f = out_ref.shape[1]
    row_a, row_c = _rows(mmax_ref, fsrc_ref)
    row_a = row_a.astype(jnp.bfloat16)
    row_c = row_c.astype(jnp.bfloat16)
    half = nbits // 2
    parts = []
    w = None
    for hi, adj_ref in ((0, adjl_ref), (1, adjr_ref)):
        adj = adj_ref[...]
        off = hi * half * cw
        t1 = row_a + d1_ref[:, off:off + half * cw].astype(jnp.bfloat16)
        t2 = row_c + d2_ref[:, off:off + half * cw].astype(jnp.bfloat16)
        t3 = jnp.maximum(t1, t2)
        t3 = jnp.where(adj > 0, t3, jnp.bfloat16(-1e9))
        parts.append(jnp.exp2(t3))
        for s in range(half):
            t = jnp.left_shift(adj[:, s * cw:(s + 1) * cw], s + hi * half)
            w = t if w is None else w | t
    p = jnp.concatenate(parts, axis=1)
    acc = jnp.dot(p, whp_ref[...], preferred_element_type=jnp.float32)
    out_ref[...] = acc[:, :f] / acc[:, f:f + 1]
    pk_ref[...] = w


def _attn_packed_body(mmax_ref, pk_ref, whp_ref, fsrc_ref, d1_ref, d2_ref,
                      out_ref, *, nbits, cw):
    f = out_ref.shape[1]
    row_a, row_c = _rows(mmax_ref, fsrc_ref)
    row_a = row_a.astype(jnp.bfloat16)
    row_c = row_c.astype(jnp.bfloat16)
    w = pk_ref[...]
    parts = []
    for s in range(nbits):
        t1 = row_a + d1_ref[:, s * cw:(s + 1) * cw].astype(jnp.bfloat16)
        t2 = row_c + d2_ref[:, s * cw:(s + 1) * cw].astype(jnp.bfloat16)
        t3 = jnp.maximum(t1, t2)
        t3 = jnp.where((w & (1 << s)) > 0, t3, jnp.bfloat16(-1e9))
        parts.append(jnp.exp2(t3))
    p = jnp.concatenate(parts, axis=1)
    acc = jnp.dot(p, whp_ref[...], preferred_element_type=jnp.float32)
    out_ref[...] = acc[:, :f] / acc[:, f:f + 1]


_NBITS = 16


def _attention(adj_or_pk, whp, fsrc, d1, d2, mmax, n, bi, packed):
    f = whp.shape[1] // 2
    npad = whp.shape[0]
    cw = npad // _NBITS
    if packed:
        body = functools.partial(_attn_packed_body, nbits=_NBITS, cw=cw)
        adj_specs = [pl.BlockSpec((bi, cw), lambda i: (i, 0))]
        adj_args = (adj_or_pk,)
        out_shape = jax.ShapeDtypeStruct((n, f), jnp.float32)
        out_specs = pl.BlockSpec((bi, f), lambda i: (i, 0))
    else:
        body = functools.partial(_attn_pack_body, nbits=_NBITS, cw=cw)
        adj_specs = [pl.BlockSpec((bi, npad // 2), lambda i: (i, 0)),
                     pl.BlockSpec((bi, npad // 2), lambda i: (i, 1))]
        adj_args = (adj_or_pk, adj_or_pk)
        out_shape = [jax.ShapeDtypeStruct((n, f), jnp.float32),
                     jax.ShapeDtypeStruct((n, cw), jnp.int32)]
        out_specs = [pl.BlockSpec((bi, f), lambda i: (i, 0)),
                     pl.BlockSpec((bi, cw), lambda i: (i, 0))]
    return pl.pallas_call(
        body,
        grid=(n // bi,),
        in_specs=[pl.BlockSpec(memory_space=pltpu.SMEM)] + adj_specs + [
            pl.BlockSpec((npad, 2 * f), lambda i: (0, 0)),
            pl.BlockSpec((bi, 1), lambda i: (i, 0)),
            pl.BlockSpec((1, npad), lambda i: (0, 0)),
            pl.BlockSpec((1, npad), lambda i: (0, 0)),
        ],
        out_specs=out_specs,
        out_shape=out_shape,
        compiler_params=pltpu.CompilerParams(
            dimension_semantics=("arbitrary",)),
    )(mmax, *adj_args, whp, fsrc, d1, d2)


def _pick(n, prefs):
    for p in prefs:
        if n % p == 0:
            return p
    return n


def kernel(x, adj, W1, a1_src, a1_dst, W2, a2_src, a2_dst):
    n, _ = x.shape
    bi = _pick(n, (400, 200, 100))
    npad = -(-n // (128 * _NBITS)) * (128 * _NBITS)
    ba = 2048 if npad % 2048 == 0 else npad

    a1s = a1_src.reshape(-1, 1).astype(jnp.float32)
    a1d = a1_dst.reshape(-1, 1).astype(jnp.float32)
    a2s = a2_src.reshape(-1, 1).astype(jnp.float32)
    a2d = a2_dst.reshape(-1, 1).astype(jnp.float32)

    whp, fs, d1, d2, mm = _project(x, W1, a1s, a1d, False, ba, npad)
    h1, pk = _attention(adj, whp, fs, d1, d2, mm, n, bi, packed=False)
    whp, fs, d1, d2, mm = _project(h1, W2, a2s, a2d, True, ba, npad)
    return _attention(pk, whp, fs, d1, d2, mm, n, bi, packed=True)


# speedup vs baseline: 1.3707x; 1.0858x over previous
import functools

import jax
import jax.numpy as jnp
from jax.experimental import pallas as pl
from jax.experimental.pallas import tpu as pltpu

_LOG2E = 1.4426950408889634
_NBITS = 16


def _proj_prologue(h_ref, w_ref, asrc_ref, adst_ref,
                   whp_s, fsrc_s, d1_s, d2_s, mmax_s, apply_elu, n, npad):
    h = h_ref[...]
    if apply_elu:
        h = jnp.where(h > 0, h, jnp.exp(h) - jnp.float32(1.0))
    wh = jnp.dot(h, w_ref[...], preferred_element_type=jnp.float32)
    f = wh.shape[1]
    whp = jnp.concatenate(
        [wh, jnp.zeros((npad - n, f), jnp.float32)], axis=0)
    rmask = jax.lax.broadcasted_iota(jnp.int32, (npad, 1), 0) < n
    ones_col = jax.lax.broadcasted_iota(jnp.int32, (1, f), 1) == 0
    ones = jnp.where(jnp.logical_and(rmask, ones_col),
                     jnp.float32(1.0), jnp.float32(0.0))
    whp_s[...] = jnp.concatenate(
        [whp.astype(jnp.bfloat16), ones.astype(jnp.bfloat16)], axis=1)
    fsrc_s[...] = jnp.dot(whp, asrc_ref[...],
                          preferred_element_type=jnp.float32)
    fd = jax.lax.dot_general(adst_ref[...], whp, (((0,), (1,)), ((), ())),
                             preferred_element_type=jnp.float32)
    cmask = jax.lax.broadcasted_iota(jnp.int32, (1, npad), 1) < n
    fd = jnp.where(cmask, fd, jnp.float32(-1e9))
    d1_s[...] = jnp.float32(_LOG2E) * fd
    d2_s[...] = jnp.float32(0.2 * _LOG2E) * fd
    mmax_s[0, 0] = jnp.max(fd)


def _rows(mmax, fsrc):
    c = jnp.float32(_LOG2E)
    v = fsrc + mmax
    cm = c * jnp.maximum(v, jnp.float32(0.2) * v)
    row_a = (c * fsrc - cm).astype(jnp.bfloat16)
    row_c = (jnp.float32(0.2) * c * fsrc - cm).astype(jnp.bfloat16)
    return row_a, row_c


def _l1_body(x_ref, w_ref, asrc_ref, adst_ref, adjl_ref, adjr_ref,
             h1_ref, pk_ref, whp_s, fsrc_s, d1_s, d2_s, mmax_s,
             *, nbits, cw, n, npad, bi):
    i = pl.program_id(0)

    @pl.when(i == 0)
    def _proj():
        _proj_prologue(x_ref, w_ref, asrc_ref, adst_ref,
                       whp_s, fsrc_s, d1_s, d2_s, mmax_s, False, n, npad)

    f = h1_ref.shape[1]
    fsrc = fsrc_s[pl.ds(i * bi, bi), :]
    row_a, row_c = _rows(mmax_s[0, 0], fsrc)
    half = nbits // 2
    parts = []
    w = None
    for hi, adj_ref in ((0, adjl_ref), (1, adjr_ref)):
        adj = adj_ref[...]
        off = hi * half * cw
        t1 = row_a + d1_s[:, off:off + half * cw].astype(jnp.bfloat16)
        t2 = row_c + d2_s[:, off:off + half * cw].astype(jnp.bfloat16)
        t3 = jnp.maximum(t1, t2)
        t3 = jnp.where(adj > 0, t3, jnp.bfloat16(-1e9))
        parts.append(jnp.exp2(t3))
        for s in range(half):
            t = jnp.left_shift(adj[:, s * cw:(s + 1) * cw], s + hi * half)
            w = t if w is None else w | t
    p = jnp.concatenate(parts, axis=1)
    acc = jnp.dot(p, whp_s[...], preferred_element_type=jnp.float32)
    h1_ref[...] = acc[:, :f] / acc[:, f:f + 1]
    pk_ref[...] = w


def _l2_body(h1_ref, w_ref, asrc_ref, adst_ref, pk_ref,
             out_ref, whp_s, fsrc_s, d1_s, d2_s, mmax_s,
             *, nbits, cw, n, npad, bi):
    i = pl.program_id(0)

    @pl.when(i == 0)
    def _proj():
        _proj_prologue(h1_ref, w_ref, asrc_ref, adst_ref,
                       whp_s, fsrc_s, d1_s, d2_s, mmax_s, True, n, npad)

    f = out_ref.shape[1]
    fsrc = fsrc_s[pl.ds(i * bi, bi), :]
    row_a, row_c = _rows(mmax_s[0, 0], fsrc)
    w = pk_ref[...]
    parts = []
    for s in range(nbits):
        t1 = row_a + d1_s[:, s * cw:(s + 1) * cw].astype(jnp.bfloat16)
        t2 = row_c + d2_s[:, s * cw:(s + 1) * cw].astype(jnp.bfloat16)
        t3 = jnp.maximum(t1, t2)
        t3 = jnp.where((w & (1 << s)) > 0, t3, jnp.bfloat16(-1e9))
        parts.append(jnp.exp2(t3))
    p = jnp.concatenate(parts, axis=1)
    acc = jnp.dot(p, whp_s[...], preferred_element_type=jnp.float32)
    out_ref[...] = acc[:, :f] / acc[:, f:f + 1]


def _scratch(npad, f):
    return [
        pltpu.VMEM((npad, 2 * f), jnp.bfloat16),
        pltpu.VMEM((npad, 1), jnp.float32),
        pltpu.VMEM((1, npad), jnp.float32),
        pltpu.VMEM((1, npad), jnp.float32),
        pltpu.SMEM((1, 1), jnp.float32),
    ]


def _pick(n, prefs):
    for p in prefs:
        if n % p == 0:
            return p
    return n


def kernel(x, adj, W1, a1_src, a1_dst, W2, a2_src, a2_dst):
    n, f = x.shape
    bi = _pick(n, (400, 200, 100))
    npad = -(-n // (128 * _NBITS)) * (128 * _NBITS)
    cw = npad // _NBITS

    a1s = a1_src.reshape(-1, 1).astype(jnp.float32)
    a1d = a1_dst.reshape(-1, 1).astype(jnp.float32)
    a2s = a2_src.reshape(-1, 1).astype(jnp.float32)
    a2d = a2_dst.reshape(-1, 1).astype(jnp.float32)

    fixed = dict(nbits=_NBITS, cw=cw, n=n, npad=npad, bi=bi)
    const = lambda a: (0, 0)

    h1, pk = pl.pallas_call(
        functools.partial(_l1_body, **fixed),
        grid=(n // bi,),
        in_specs=[
            pl.BlockSpec((n, f), const),
            pl.BlockSpec((f, f), const),
            pl.BlockSpec((f, 1), const),
            pl.BlockSpec((f, 1), const),
            pl.BlockSpec((bi, npad // 2), lambda i: (i, 0)),
            pl.BlockSpec((bi, npad // 2), lambda i: (i, 1)),
        ],
        out_specs=[pl.BlockSpec((bi, f), lambda i: (i, 0)),
                   pl.BlockSpec((bi, cw), lambda i: (i, 0))],
        out_shape=[jax.ShapeDtypeStruct((n, f), jnp.float32),
                   jax.ShapeDtypeStruct((n, cw), jnp.int32)],
        scratch_shapes=_scratch(npad, f),
        compiler_params=pltpu.CompilerParams(
            dimension_semantics=("arbitrary",)),
    )(x, W1, a1s, a1d, adj, adj)

    return pl.pallas_call(
        functools.partial(_l2_body, **fixed),
        grid=(n // bi,),
        in_specs=[
            pl.BlockSpec((n, f), const),
            pl.BlockSpec((f, f), const),
            pl.BlockSpec((f, 1), const),
            pl.BlockSpec((f, 1), const),
            pl.BlockSpec((bi, cw), lambda i: (i, 0)),
        ],
        out_specs=pl.BlockSpec((bi, f), lambda i: (i, 0)),
        out_shape=jax.ShapeDtypeStruct((n, f), jnp.float32),
        scratch_shapes=_scratch(npad, f),
        compiler_params=pltpu.CompilerParams(
            dimension_semantics=("arbitrary",)),
    )(h1, W2, a2s, a2d, pk)
